# baseline (speedup 1.0000x reference)
import functools

import jax
import jax.numpy as jnp
from jax import lax
from jax.experimental import pallas as pl
from jax.experimental.pallas import tpu as pltpu
from jax.experimental.pallas import tpu_sc as plsc

DUSER = 100000
DITEM = 1000000
DEMB = 64
DHIDDEN = 256
BATCH = 4096

_NC = 2
_NS = 16
_NW = _NC * _NS
_BPW = BATCH // _NW
_RB = 32
_NR = _BPW // _RB
_L = 16


def _extract_rows(tiles, sub_ref, emb_ref, r):
    lane = lax.iota(jnp.int32, _L)
    for j in range(_RB):
        i = r * _RB + j
        splat = plsc.load_gather(sub_ref, [jnp.full((_L,), i, jnp.int32)])
        row = jnp.full((_L,), j, jnp.int32)
        for c in range(DEMB // _L):
            val = plsc.load_gather(tiles, [row, splat, lane + (c * _L)])
            emb_ref[i, pl.ds(c * _L, _L)] = val


def _gather_one_table(table3, ids_hbm, out_hbm, base, idx_v, tid_v, sub_v,
                      buf0, buf1, emb_v, sem0, sem1):
    pltpu.sync_copy(ids_hbm.at[pl.ds(base, _BPW)], idx_v)
    lane = lax.iota(jnp.int32, _L)
    copies = []
    for c in range(_BPW // _L):
        chunk = idx_v[pl.ds(c * _L, _L)]
        for j in range(_L):
            rid = jnp.sum(jnp.where(lane == j, chunk, 0))
            i = c * _L + j
            copies.append(pltpu.async_copy(
                table3.at[pl.ds(rid, 1)], emb_v.at[pl.ds(i, 1)],
                sem0))
    for cp in copies:
        cp.wait()
    pltpu.sync_copy(emb_v, out_hbm.at[pl.ds(base, _BPW)])


def _sc_gather_body(user_t3, item_t3, uid, iid, u_out, i_out,
                    idx_v, tid_v, sub_v, buf0, buf1, uemb_v, iemb_v,
                    sem0, sem1):
    wid = lax.axis_index("s") * _NC + lax.axis_index("c")
    base = wid * _BPW
    _gather_one_table(user_t3, uid, u_out, base, idx_v, tid_v, sub_v,
                      buf0, buf1, uemb_v, sem0, sem1)
    _gather_one_table(item_t3, iid, i_out, base, idx_v, tid_v, sub_v,
                      buf0, buf1, iemb_v, sem0, sem1)


@functools.cache
def _sc_gather():
    return pl.kernel(
        _sc_gather_body,
        out_type=[
            jax.ShapeDtypeStruct((BATCH, DEMB), jnp.float32),
            jax.ShapeDtypeStruct((BATCH, DEMB), jnp.float32),
        ],
        mesh=plsc.VectorSubcoreMesh(
            core_axis_name="c", subcore_axis_name="s",
            num_cores=_NC, num_subcores=_NS),
        compiler_params=pltpu.CompilerParams(needs_layout_passes=False),
        scratch_types=[
            pltpu.VMEM((_BPW,), jnp.int32),
            pltpu.VMEM((_NR, _RB), jnp.int32),
            pltpu.VMEM((_BPW,), jnp.int32),
            pltpu.VMEM((_RB, 8, DEMB), jnp.float32),
            pltpu.VMEM((_RB, 8, DEMB), jnp.float32),
            pltpu.VMEM((_BPW, DEMB), jnp.float32),
            pltpu.VMEM((_BPW, DEMB), jnp.float32),
            pltpu.SemaphoreType.DMA,
            pltpu.SemaphoreType.DMA,
        ],
    )


def _mlp_body(u_ref, i_ref, w1u_ref, w1i_ref, b1_ref, w2_ref, b2_ref,
              w3_ref, b3_ref, out_ref):
    h = (jnp.dot(u_ref[...], w1u_ref[...], preferred_element_type=jnp.float32)
         + jnp.dot(i_ref[...], w1i_ref[...], preferred_element_type=jnp.float32)
         + b1_ref[...])
    h = jax.nn.sigmoid(h)
    h = jax.nn.sigmoid(
        jnp.dot(h, w2_ref[...], preferred_element_type=jnp.float32)
        + b2_ref[...])
    out_ref[...] = jax.nn.sigmoid(
        jnp.dot(h, w3_ref[...], preferred_element_type=jnp.float32)
        + b3_ref[...])


def _mlp(u_emb, i_emb, w1u, w1i, b1, w2, b2, w3, b3, block_b=512):
    grid = (BATCH // block_b,)
    full = lambda *s: pl.BlockSpec(s, lambda j: (0,) * len(s))
    return pl.pallas_call(
        _mlp_body,
        grid=grid,
        in_specs=[
            pl.BlockSpec((block_b, DEMB), lambda j: (j, 0)),
            pl.BlockSpec((block_b, DEMB), lambda j: (j, 0)),
            full(DEMB, DHIDDEN),
            full(DEMB, DHIDDEN),
            full(1, DHIDDEN),
            full(DHIDDEN, DHIDDEN),
            full(1, DHIDDEN),
            full(DHIDDEN, 1),
            full(1, 1),
        ],
        out_specs=pl.BlockSpec((block_b, 1), lambda j: (j, 0)),
        out_shape=jax.ShapeDtypeStruct((BATCH, 1), jnp.float32),
    )(u_emb, i_emb, w1u, w1i, b1, w2, b2, w3, b3)


def kernel(user_id, item_id, user_table, item_table, W1, b1, W2, b2, W3, b3):
    u_emb, i_emb = _sc_gather()(
        user_table, item_table,
        user_id.astype(jnp.int32),
        item_id.astype(jnp.int32))
    return _mlp(u_emb, i_emb,
                W1[:DEMB], W1[DEMB:],
                b1.reshape(1, DHIDDEN), W2, b2.reshape(1, DHIDDEN),
                W3, b3.reshape(1, 1))
